# baseline (speedup 1.0000x reference)
import functools

import jax
import jax.numpy as jnp
from jax import lax
from jax.experimental import pallas as pl
from jax.experimental.pallas import tpu as pltpu

KTOP = 64
T_TOK = 2048
D_MODEL = 1024
D_SAE = 16384

ENC_CH = 512
DEC_CH = 1024
THR_TB = 128
MAX_FINITE_BITS = 0x7F7FFFFF


def _encode_body(x_ref, w_ref, b_ref, z_ref, zt_ref, s8_ref, l8_ref):
    acc = jnp.dot(x_ref[...].astype(jnp.bfloat16),
                  w_ref[...].astype(jnp.bfloat16),
                  preferred_element_type=jnp.float32)
    z = jnp.maximum(acc + b_ref[...], 0.0)
    z_ref[...] = z
    bits = lax.bitcast_convert_type(z, jnp.int32)
    zt_ref[...] = lax.bitcast_convert_type((bits >> 16).astype(jnp.uint16),
                                           jnp.bfloat16)
    s8_ref[...] = ((bits >> 8) & 0xFF).astype(jnp.bfloat16)
    l8_ref[...] = (bits & 0xFF).astype(jnp.bfloat16)


_CCH = 256


def _count_cmp(arr_bf, thr_bf, strict=False):
    one = jnp.ones((), jnp.bfloat16)
    zero = jnp.zeros((), jnp.bfloat16)
    acc = jnp.zeros((THR_TB, _CCH), jnp.bfloat16)
    for c in range(D_SAE // _CCH):
        s = arr_bf[:, c * _CCH:(c + 1) * _CCH]
        cond = s > thr_bf if strict else s >= thr_bf
        acc = acc + jnp.where(cond, one, zero)
    return jnp.sum(acc.astype(jnp.float32), axis=1,
                   keepdims=True).astype(jnp.int32)


def _bisect(arr_bf, lo0, hi0, need, n_iter, to_bf):
    def step(_, carry):
        lo, hi = carry
        mid = lo + ((hi - lo + 1) >> 1)
        take = _count_cmp(arr_bf, to_bf(mid)) >= need
        return jnp.where(take, mid, lo), jnp.where(take, hi, mid - 1)
    lo, _ = lax.fori_loop(0, n_iter, step, (lo0, hi0))
    return lo


def _to_bf_bits(m):
    return lax.bitcast_convert_type(m.astype(jnp.uint16), jnp.bfloat16)


def _to_bf_int(m):
    return m.astype(jnp.bfloat16)


def _threshold_body(zt_ref, s8_ref, l8_ref, thr_ref):
    zt = zt_ref[...]
    s8 = s8_ref[...]
    l8 = l8_ref[...]
    neg1 = jnp.full((), -1.0, jnp.bfloat16)

    zeros = jnp.zeros((THR_TB, 1), jnp.int32)
    kvec = jnp.full((THR_TB, 1), KTOP, jnp.int32)
    hi16 = jnp.full((THR_TB, 1), MAX_FINITE_BITS >> 16, jnp.int32)
    c255 = jnp.full((THR_TB, 1), 255, jnp.int32)

    t16 = _bisect(zt, zeros, hi16, kvec, 15, _to_bf_bits)
    t16_bf = _to_bf_bits(t16)

    need_a = kvec - _count_cmp(zt, t16_bf, strict=True)
    key_a = jnp.where(zt == t16_bf, s8, neg1)
    t8 = _bisect(key_a, zeros, c255, need_a, 8, _to_bf_int)
    t8_bf = _to_bf_int(t8)

    need_b = need_a - _count_cmp(key_a, t8_bf, strict=True)
    key_b = jnp.where((zt == t16_bf) & (s8 == t8_bf), l8, neg1)
    tl8 = _bisect(key_b, zeros, c255, need_b, 8, _to_bf_int)

    thr_bits = (t16 << 16) | (t8 << 8) | tl8
    thr_ref[...] = jnp.broadcast_to(thr_bits, (THR_TB, 128))


def _decode_body(z_ref, thr_ref, w_ref, b_ref, out_ref):
    c = pl.program_id(0)
    thr = lax.bitcast_convert_type(thr_ref[:, 0:1], jnp.float32)
    zb = z_ref[...]
    zs = jnp.where(zb >= thr, zb, 0.0)
    partial = jnp.dot(zs.astype(jnp.bfloat16),
                      w_ref[...].astype(jnp.bfloat16),
                      preferred_element_type=jnp.float32)

    @pl.when(c == 0)
    def _():
        out_ref[...] = partial + b_ref[...]

    @pl.when(c != 0)
    def _():
        out_ref[...] += partial


def kernel(x, W_enc, b_enc, W_dec, b_dec):
    n_enc = D_SAE // ENC_CH
    zchunk = pl.BlockSpec((T_TOK, ENC_CH), lambda c: (0, c))
    z, zt, s8, l8 = pl.pallas_call(
        _encode_body,
        grid=(n_enc,),
        in_specs=[
            pl.BlockSpec((T_TOK, D_MODEL), lambda c: (0, 0)),
            pl.BlockSpec((D_MODEL, ENC_CH), lambda c: (0, c)),
            pl.BlockSpec((1, ENC_CH), lambda c: (0, c)),
        ],
        out_specs=[zchunk, zchunk, zchunk, zchunk],
        out_shape=[
            jax.ShapeDtypeStruct((T_TOK, D_SAE), jnp.float32),
            jax.ShapeDtypeStruct((T_TOK, D_SAE), jnp.bfloat16),
            jax.ShapeDtypeStruct((T_TOK, D_SAE), jnp.bfloat16),
            jax.ShapeDtypeStruct((T_TOK, D_SAE), jnp.bfloat16),
        ],
        compiler_params=pltpu.CompilerParams(
            dimension_semantics=("arbitrary",)),
    )(x, W_enc, b_enc.reshape(1, D_SAE))

    n_tb = T_TOK // THR_TB
    tblk = pl.BlockSpec((THR_TB, D_SAE), lambda t: (t, 0))
    thr = pl.pallas_call(
        _threshold_body,
        grid=(n_tb,),
        in_specs=[tblk, tblk, tblk],
        out_specs=pl.BlockSpec((THR_TB, 128), lambda t: (t, 0)),
        out_shape=jax.ShapeDtypeStruct((T_TOK, 128), jnp.int32),
        compiler_params=pltpu.CompilerParams(
            dimension_semantics=("arbitrary",)),
    )(zt, s8, l8)

    n_dec = D_SAE // DEC_CH
    out = pl.pallas_call(
        _decode_body,
        grid=(n_dec,),
        in_specs=[
            pl.BlockSpec((T_TOK, DEC_CH), lambda c: (0, c)),
            pl.BlockSpec((T_TOK, 128), lambda c: (0, 0)),
            pl.BlockSpec((DEC_CH, D_MODEL), lambda c: (c, 0)),
            pl.BlockSpec((1, D_MODEL), lambda c: (0, 0)),
        ],
        out_specs=pl.BlockSpec((T_TOK, D_MODEL), lambda c: (0, 0)),
        out_shape=jax.ShapeDtypeStruct((T_TOK, D_MODEL), jnp.float32),
        compiler_params=pltpu.CompilerParams(
            dimension_semantics=("arbitrary",)),
    )(z, thr, W_dec, b_dec.reshape(1, D_MODEL))
    return out


# speedup vs baseline: 1.1743x; 1.1743x over previous
import functools

import jax
import jax.numpy as jnp
from jax import lax
from jax.experimental import pallas as pl
from jax.experimental.pallas import tpu as pltpu

KTOP = 64
T_TOK = 2048
D_MODEL = 1024
D_SAE = 16384

ENC_CH = 1024
DEC_CH = 1024
THR_TB = 128
MAX_FINITE_BITS = 0x7F7FFFFF


def _encode_body(x_ref, w_ref, b_ref, z_ref):
    acc = jnp.dot(x_ref[...].astype(jnp.bfloat16),
                  w_ref[...].astype(jnp.bfloat16),
                  preferred_element_type=jnp.float32)
    z_ref[...] = jnp.maximum(acc + b_ref[...], 0.0)


_CCH = 256


def _count_cmp(arr_bf, thr_bf, strict=False):
    one = jnp.ones((), jnp.bfloat16)
    zero = jnp.zeros((), jnp.bfloat16)
    acc = jnp.zeros((THR_TB, _CCH), jnp.bfloat16)
    for c in range(D_SAE // _CCH):
        s = arr_bf[:, c * _CCH:(c + 1) * _CCH]
        cond = s > thr_bf if strict else s >= thr_bf
        acc = acc + jnp.where(cond, one, zero)
    return jnp.sum(acc.astype(jnp.float32), axis=1,
                   keepdims=True).astype(jnp.int32)


def _bisect(arr_bf, lo0, hi0, need, n_iter, to_bf):
    def step(_, carry):
        lo, hi = carry
        mid = lo + ((hi - lo + 1) >> 1)
        take = _count_cmp(arr_bf, to_bf(mid)) >= need
        return jnp.where(take, mid, lo), jnp.where(take, hi, mid - 1)
    lo, _ = lax.fori_loop(0, n_iter, step, (lo0, hi0))
    return lo


def _to_bf_bits(m):
    return lax.bitcast_convert_type(m.astype(jnp.uint16), jnp.bfloat16)


def _to_bf_int(m):
    return m.astype(jnp.bfloat16)


def _threshold_body(z_ref, thr_ref):
    bits = lax.bitcast_convert_type(z_ref[...], jnp.int32)
    zt = _to_bf_bits(bits >> 16)
    s8 = ((bits >> 8) & 0xFF).astype(jnp.bfloat16)
    l8 = (bits & 0xFF).astype(jnp.bfloat16)
    neg1 = jnp.full((), -1.0, jnp.bfloat16)

    zeros = jnp.zeros((THR_TB, 1), jnp.int32)
    kvec = jnp.full((THR_TB, 1), KTOP, jnp.int32)
    c255 = jnp.full((THR_TB, 1), 255, jnp.int32)

    cm = zt[:, 0:_CCH]
    for c in range(1, D_SAE // _CCH):
        cm = jnp.maximum(cm, zt[:, c * _CCH:(c + 1) * _CCH])
    cmf = cm.astype(jnp.float32)
    lo0 = (lax.bitcast_convert_type(
        jnp.min(cmf, axis=1, keepdims=True), jnp.int32) >> 16)
    hi0 = (lax.bitcast_convert_type(
        jnp.max(cmf, axis=1, keepdims=True), jnp.int32) >> 16)

    def s1_cond(carry):
        lo, hi = carry
        return jnp.max(hi - lo) > 0

    def s1_body(carry):
        lo, hi = carry
        mid = lo + ((hi - lo + 1) >> 1)
        take = _count_cmp(zt, _to_bf_bits(mid)) >= kvec
        return jnp.where(take, mid, lo), jnp.where(take, hi, mid - 1)

    t16, _ = lax.while_loop(s1_cond, s1_body, (lo0, hi0))
    t16_bf = _to_bf_bits(t16)

    need_a = kvec - _count_cmp(zt, t16_bf, strict=True)
    key_a = jnp.where(zt == t16_bf, s8, neg1)
    t8 = _bisect(key_a, zeros, c255, need_a, 8, _to_bf_int)
    t8_bf = _to_bf_int(t8)

    need_b = need_a - _count_cmp(key_a, t8_bf, strict=True)
    key_b = jnp.where((zt == t16_bf) & (s8 == t8_bf), l8, neg1)
    tl8 = _bisect(key_b, zeros, c255, need_b, 8, _to_bf_int)

    thr_bits = (t16 << 16) | (t8 << 8) | tl8
    thr_ref[...] = jnp.broadcast_to(thr_bits, (THR_TB, 128))


def _decode_body(z_ref, thr_ref, w_ref, b_ref, out_ref):
    c = pl.program_id(0)
    thr = lax.bitcast_convert_type(thr_ref[:, 0:1], jnp.float32)
    zb = z_ref[...]
    zs = jnp.where(zb >= thr, zb, 0.0)
    partial = jnp.dot(zs.astype(jnp.bfloat16),
                      w_ref[...].astype(jnp.bfloat16),
                      preferred_element_type=jnp.float32)

    @pl.when(c == 0)
    def _():
        out_ref[...] = partial + b_ref[...]

    @pl.when(c != 0)
    def _():
        out_ref[...] += partial


def kernel(x, W_enc, b_enc, W_dec, b_dec):
    n_enc = D_SAE // ENC_CH
    z = pl.pallas_call(
        _encode_body,
        grid=(n_enc,),
        in_specs=[
            pl.BlockSpec((T_TOK, D_MODEL), lambda c: (0, 0)),
            pl.BlockSpec((D_MODEL, ENC_CH), lambda c: (0, c)),
            pl.BlockSpec((1, ENC_CH), lambda c: (0, c)),
        ],
        out_specs=pl.BlockSpec((T_TOK, ENC_CH), lambda c: (0, c)),
        out_shape=jax.ShapeDtypeStruct((T_TOK, D_SAE), jnp.float32),
        compiler_params=pltpu.CompilerParams(
            dimension_semantics=("arbitrary",)),
    )(x, W_enc, b_enc.reshape(1, D_SAE))

    n_tb = T_TOK // THR_TB
    thr = pl.pallas_call(
        _threshold_body,
        grid=(n_tb,),
        in_specs=[pl.BlockSpec((THR_TB, D_SAE), lambda t: (t, 0))],
        out_specs=pl.BlockSpec((THR_TB, 128), lambda t: (t, 0)),
        out_shape=jax.ShapeDtypeStruct((T_TOK, 128), jnp.int32),
        compiler_params=pltpu.CompilerParams(
            dimension_semantics=("arbitrary",)),
    )(z)

    n_dec = D_SAE // DEC_CH
    out = pl.pallas_call(
        _decode_body,
        grid=(n_dec,),
        in_specs=[
            pl.BlockSpec((T_TOK, DEC_CH), lambda c: (0, c)),
            pl.BlockSpec((T_TOK, 128), lambda c: (0, 0)),
            pl.BlockSpec((DEC_CH, D_MODEL), lambda c: (c, 0)),
            pl.BlockSpec((1, D_MODEL), lambda c: (0, 0)),
        ],
        out_specs=pl.BlockSpec((T_TOK, D_MODEL), lambda c: (0, 0)),
        out_shape=jax.ShapeDtypeStruct((T_TOK, D_MODEL), jnp.float32),
        compiler_params=pltpu.CompilerParams(
            dimension_semantics=("arbitrary",)),
    )(z, thr, W_dec, b_dec.reshape(1, D_MODEL))
    return out
